# baseline (speedup 1.0000x reference)
import functools

import jax
import jax.numpy as jnp
from jax import lax
from jax.experimental import pallas as pl
from jax.experimental.pallas import tpu as pltpu
from jax.experimental.pallas import tpu_sc as plsc


def _make_sc_gather(V, D, B):
    info = plsc.get_sparse_core_info()
    NC, NS = info.num_cores, info.num_subcores
    NW = NC * NS
    assert B % (8 * NW) == 0 and D % info.num_lanes == 0
    b_per_w = B // NW
    mesh = plsc.VectorSubcoreMesh(core_axis_name="c", subcore_axis_name="s")

    @functools.partial(
        pl.kernel,
        mesh=mesh,
        compiler_params=pltpu.CompilerParams(use_tc_tiling_on_sc=False),
        out_type=jax.ShapeDtypeStruct((B, D), jnp.float32),
        scratch_types=[
            pltpu.VMEM((b_per_w,), jnp.int32),
            pltpu.VMEM((b_per_w, D), jnp.float32),
            pltpu.SemaphoreType.DMA,
        ],
    )
    def gather_k(table_hbm, idx_hbm, out_hbm, idx_v, rows_v, sem):
        wid = lax.axis_index("s") * NC + lax.axis_index("c")
        base = wid * b_per_w
        pltpu.sync_copy(idx_hbm.at[pl.ds(base, b_per_w)], idx_v)
        pltpu.async_copy(table_hbm.at[idx_v], rows_v, sem).wait()
        pltpu.sync_copy(rows_v, out_hbm.at[pl.ds(base, b_per_w)])

    return gather_k


def _mlp_body(x_ref, w1_ref, b1_ref, w2_ref, b2_ref, o_ref):
    x = x_ref[...]
    h = jnp.dot(x, w1_ref[...], preferred_element_type=jnp.float32) + b1_ref[...]
    h = jnp.maximum(h, 0.0)
    y = jnp.dot(h, w2_ref[...], preferred_element_type=jnp.float32) + b2_ref[...]
    ss = jnp.sum(y * y, axis=-1, keepdims=True)
    o_ref[...] = y / jnp.maximum(jnp.sqrt(ss), 1e-12)


def _mlp(gathered, W1, b1, W2, b2, blk=2048):
    B, D = gathered.shape
    H = W1.shape[1]
    O = W2.shape[1]
    return pl.pallas_call(
        _mlp_body,
        grid=(B // blk,),
        in_specs=[
            pl.BlockSpec((blk, D), lambda i: (i, 0)),
            pl.BlockSpec((D, H), lambda i: (0, 0)),
            pl.BlockSpec((1, H), lambda i: (0, 0)),
            pl.BlockSpec((H, O), lambda i: (0, 0)),
            pl.BlockSpec((1, O), lambda i: (0, 0)),
        ],
        out_specs=pl.BlockSpec((blk, O), lambda i: (i, 0)),
        out_shape=jax.ShapeDtypeStruct((B, O), jnp.float32),
    )(gathered, W1, b1.reshape(1, H), W2, b2.reshape(1, O))


def kernel(indices, table, W1, b1, W2, b2):
    idx = indices.astype(jnp.int32)
    B = idx.shape[0]
    V, D = table.shape
    gathered = _make_sc_gather(V, D, B)(table, idx)
    return _mlp(gathered, W1, b1, W2, b2)


# speedup vs baseline: 1.6702x; 1.6702x over previous
import functools

import jax
import jax.numpy as jnp
from jax import lax
from jax.experimental import pallas as pl
from jax.experimental.pallas import tpu as pltpu
from jax.experimental.pallas import tpu_sc as plsc


_CH = 16


def _make_sc_gather(V, D, B):
    info = plsc.get_sparse_core_info()
    NC, NS = info.num_cores, info.num_subcores
    NW = NC * NS
    assert B % (8 * NW) == 0 and D % info.num_lanes == 0
    b_per_w = B // NW
    n_chunks = b_per_w // _CH
    mesh = plsc.VectorSubcoreMesh(core_axis_name="c", subcore_axis_name="s")

    @functools.partial(
        pl.kernel,
        mesh=mesh,
        out_type=jax.ShapeDtypeStruct((B, D), jnp.float32),
        scratch_types=[
            pltpu.VMEM((b_per_w,), jnp.int32),
            pltpu.VMEM((b_per_w, D), jnp.float32),
            pltpu.SemaphoreType.DMA,
            pltpu.SemaphoreType.DMA,
        ],
    )
    def gather_k(table_hbm, idx_hbm, out_hbm, idx_s, rows_v, sem_idx, sem):
        wid = lax.axis_index("s") * NC + lax.axis_index("c")
        base = wid * b_per_w
        pltpu.async_copy(idx_hbm.at[pl.ds(base, b_per_w)], idx_s, sem_idx).wait()

        def issue(g):
            vec = idx_s[pl.ds(g * _CH, _CH)]
            for j in range(_CH):
                i = g * _CH + j
                pltpu.async_copy(
                    table_hbm.at[pl.ds(vec[j], 1)], rows_v.at[pl.ds(i, 1)], sem
                )

        def drain(g):
            for j in range(_CH):
                i = g * _CH + j
                pltpu.make_async_copy(
                    table_hbm.at[pl.ds(0, 1)], rows_v.at[pl.ds(i, 1)], sem
                ).wait()

        issue(0)

        def body(g, _):
            issue(g + 1)
            drain(g)
            return ()

        lax.fori_loop(0, n_chunks - 1, body, (), unroll=False)
        drain(n_chunks - 1)
        pltpu.sync_copy(rows_v, out_hbm.at[pl.ds(base, b_per_w)])

    return gather_k


def _mlp_body(x_ref, w1_ref, b1_ref, w2_ref, b2_ref, o_ref):
    x = x_ref[...]
    h = jnp.dot(x, w1_ref[...], preferred_element_type=jnp.float32) + b1_ref[...]
    h = jnp.maximum(h, 0.0)
    y = jnp.dot(h, w2_ref[...], preferred_element_type=jnp.float32) + b2_ref[...]
    ss = jnp.sum(y * y, axis=-1, keepdims=True)
    o_ref[...] = y / jnp.maximum(jnp.sqrt(ss), 1e-12)


def _mlp(gathered, W1, b1, W2, b2, blk=2048):
    B, D = gathered.shape
    H = W1.shape[1]
    O = W2.shape[1]
    return pl.pallas_call(
        _mlp_body,
        grid=(B // blk,),
        in_specs=[
            pl.BlockSpec((blk, D), lambda i: (i, 0)),
            pl.BlockSpec((D, H), lambda i: (0, 0)),
            pl.BlockSpec((1, H), lambda i: (0, 0)),
            pl.BlockSpec((H, O), lambda i: (0, 0)),
            pl.BlockSpec((1, O), lambda i: (0, 0)),
        ],
        out_specs=pl.BlockSpec((blk, O), lambda i: (i, 0)),
        out_shape=jax.ShapeDtypeStruct((B, O), jnp.float32),
    )(gathered, W1, b1.reshape(1, H), W2, b2.reshape(1, O))


def kernel(indices, table, W1, b1, W2, b2):
    idx = indices.astype(jnp.int32)
    B = idx.shape[0]
    V, D = table.shape
    gathered = _make_sc_gather(V, D, B)(table, idx)
    return _mlp(gathered, W1, b1, W2, b2)


# speedup vs baseline: 1.6814x; 1.0067x over previous
import functools

import jax
import jax.numpy as jnp
from jax import lax
from jax.experimental import pallas as pl
from jax.experimental.pallas import tpu as pltpu
from jax.experimental.pallas import tpu_sc as plsc


_CH = 32


def _make_sc_gather(V, D, B):
    info = plsc.get_sparse_core_info()
    NC, NS = info.num_cores, info.num_subcores
    NW = NC * NS
    assert B % (8 * NW) == 0 and D % info.num_lanes == 0
    b_per_w = B // NW
    n_chunks = b_per_w // _CH
    mesh = plsc.VectorSubcoreMesh(core_axis_name="c", subcore_axis_name="s")

    @functools.partial(
        pl.kernel,
        mesh=mesh,
        out_type=jax.ShapeDtypeStruct((B, D), jnp.float32),
        scratch_types=[
            pltpu.VMEM((b_per_w,), jnp.int32),
            pltpu.VMEM((b_per_w, D), jnp.float32),
            pltpu.SemaphoreType.DMA,
            pltpu.SemaphoreType.DMA,
        ],
    )
    def gather_k(table_hbm, idx_hbm, out_hbm, idx_s, rows_v, sem_idx, sem):
        wid = lax.axis_index("s") * NC + lax.axis_index("c")
        base = wid * b_per_w
        pltpu.async_copy(idx_hbm.at[pl.ds(base, b_per_w)], idx_s, sem_idx).wait()

        def issue(g):
            for h in range(_CH // 16):
                vec = idx_s[pl.ds(g * _CH + h * 16, 16)]
                for j in range(16):
                    i = g * _CH + h * 16 + j
                    pltpu.async_copy(
                        table_hbm.at[pl.ds(vec[j], 1)], rows_v.at[pl.ds(i, 1)], sem
                    )

        def drain(g):
            pltpu.make_async_copy(
                table_hbm.at[pl.ds(0, _CH)], rows_v.at[pl.ds(g * _CH, _CH)], sem
            ).wait()

        issue(0)

        def body(g, _):
            issue(g + 1)
            drain(g)
            return ()

        lax.fori_loop(0, n_chunks - 1, body, (), unroll=False)
        drain(n_chunks - 1)
        pltpu.sync_copy(rows_v, out_hbm.at[pl.ds(base, b_per_w)])

    return gather_k


def _mlp_body(x_ref, w1_ref, b1_ref, w2_ref, b2_ref, o_ref):
    x = x_ref[...]
    h = jnp.dot(x, w1_ref[...], preferred_element_type=jnp.float32) + b1_ref[...]
    h = jnp.maximum(h, 0.0)
    y = jnp.dot(h, w2_ref[...], preferred_element_type=jnp.float32) + b2_ref[...]
    ss = jnp.sum(y * y, axis=-1, keepdims=True)
    o_ref[...] = y / jnp.maximum(jnp.sqrt(ss), 1e-12)


def _mlp(gathered, W1, b1, W2, b2, blk=2048):
    B, D = gathered.shape
    H = W1.shape[1]
    O = W2.shape[1]
    return pl.pallas_call(
        _mlp_body,
        grid=(B // blk,),
        in_specs=[
            pl.BlockSpec((blk, D), lambda i: (i, 0)),
            pl.BlockSpec((D, H), lambda i: (0, 0)),
            pl.BlockSpec((1, H), lambda i: (0, 0)),
            pl.BlockSpec((H, O), lambda i: (0, 0)),
            pl.BlockSpec((1, O), lambda i: (0, 0)),
        ],
        out_specs=pl.BlockSpec((blk, O), lambda i: (i, 0)),
        out_shape=jax.ShapeDtypeStruct((B, O), jnp.float32),
    )(gathered, W1, b1.reshape(1, H), W2, b2.reshape(1, O))


def kernel(indices, table, W1, b1, W2, b2):
    idx = indices.astype(jnp.int32)
    B = idx.shape[0]
    V, D = table.shape
    gathered = _make_sc_gather(V, D, B)(table, idx)
    return _mlp(gathered, W1, b1, W2, b2)


# speedup vs baseline: 2.3518x; 1.3987x over previous
import functools

import jax
import jax.numpy as jnp
from jax import lax
from jax.experimental import pallas as pl
from jax.experimental.pallas import tpu as pltpu
from jax.experimental.pallas import tpu_sc as plsc

_CH = 32
_CB = 8192
_V1 = 507904
_P = 499712


def _repack_body(xa_ref, xb_ref, o_ref):
    xa = xa_ref[...]
    xb = xb_ref[...]
    o_ref[...] = jnp.concatenate([xa.T, xb.T], axis=1)


def _repack(tableT):
    D, V = tableT.shape
    grid = _V1 // _CB
    return pl.pallas_call(
        _repack_body,
        grid=(grid,),
        in_specs=[
            pl.BlockSpec((D, _CB), lambda i: (0, i)),
            pl.BlockSpec((D, _CB), lambda i: (0, i + _P // _CB)),
        ],
        out_specs=pl.BlockSpec((_CB, 2 * D), lambda i: (i, 0)),
        out_shape=jax.ShapeDtypeStruct((_V1, 2 * D), jnp.float32),
    )(tableT, tableT)


def _make_sc_gather(V2, D2, B):
    info = plsc.get_sparse_core_info()
    NC, NS = info.num_cores, info.num_subcores
    NW = NC * NS
    assert B % (8 * NW) == 0 and D2 % info.num_lanes == 0
    b_per_w = B // NW
    mesh = plsc.VectorSubcoreMesh(core_axis_name="c", subcore_axis_name="s")

    @functools.partial(
        pl.kernel,
        mesh=mesh,
        out_type=jax.ShapeDtypeStruct((B, D2), jnp.float32),
        scratch_types=[
            pltpu.VMEM((b_per_w,), jnp.int32),
            pltpu.VMEM((b_per_w, D2), jnp.float32),
            pltpu.SemaphoreType.DMA,
            pltpu.SemaphoreType.DMA,
        ],
    )
    def gather_k(table_hbm, idx_hbm, out_hbm, idx_v, rows_v, sem_idx, sem):
        wid = lax.axis_index("s") * NC + lax.axis_index("c")
        base = wid * b_per_w
        n_chunks = b_per_w // _CH
        pltpu.async_copy(idx_hbm.at[pl.ds(base, b_per_w)], idx_v, sem_idx).wait()

        def issue(g):
            for h in range(_CH // 16):
                vec = idx_v[pl.ds(g * _CH + h * 16, 16)]
                for j in range(16):
                    i = g * _CH + h * 16 + j
                    pltpu.async_copy(
                        table_hbm.at[pl.ds(vec[j], 1)], rows_v.at[pl.ds(i, 1)], sem
                    )

        def drain(g):
            pltpu.make_async_copy(
                table_hbm.at[pl.ds(0, _CH)], rows_v.at[pl.ds(g * _CH, _CH)], sem
            ).wait()

        issue(0)

        def body(g, _):
            issue(g + 1)
            drain(g)
            return ()

        lax.fori_loop(0, n_chunks - 1, body, (), unroll=False)
        drain(n_chunks - 1)
        pltpu.sync_copy(rows_v, out_hbm.at[pl.ds(base, b_per_w)])

    return gather_k


def _mlp_body(x2_ref, par_ref, w1_ref, b1_ref, w2_ref, b2_ref, o_ref):
    x2 = x2_ref[...]
    par = par_ref[...]
    D = w1_ref.shape[0]
    x = jnp.where(par == 0, x2[:, :D], x2[:, D:])
    h = jnp.dot(x, w1_ref[...], preferred_element_type=jnp.float32) + b1_ref[...]
    h = jnp.maximum(h, 0.0)
    y = jnp.dot(h, w2_ref[...], preferred_element_type=jnp.float32) + b2_ref[...]
    ss = jnp.sum(y * y, axis=-1, keepdims=True)
    o_ref[...] = y / jnp.maximum(jnp.sqrt(ss), 1e-12)


def _mlp(gathered2, parity, W1, b1, W2, b2, blk=2048):
    B, D2 = gathered2.shape
    D = D2 // 2
    H = W1.shape[1]
    O = W2.shape[1]
    return pl.pallas_call(
        _mlp_body,
        grid=(B // blk,),
        in_specs=[
            pl.BlockSpec((blk, D2), lambda i: (i, 0)),
            pl.BlockSpec((blk, 1), lambda i: (i, 0)),
            pl.BlockSpec((D, H), lambda i: (0, 0)),
            pl.BlockSpec((1, H), lambda i: (0, 0)),
            pl.BlockSpec((H, O), lambda i: (0, 0)),
            pl.BlockSpec((1, O), lambda i: (0, 0)),
        ],
        out_specs=pl.BlockSpec((blk, O), lambda i: (i, 0)),
        out_shape=jax.ShapeDtypeStruct((B, O), jnp.float32),
    )(gathered2, parity, W1, b1.reshape(1, H), W2, b2.reshape(1, O))


def kernel(indices, table, W1, b1, W2, b2):
    idx = indices.astype(jnp.int32)
    B = idx.shape[0]
    V, D = table.shape
    t2 = _repack(table.T)
    idx2 = jnp.where(idx < _V1, idx, idx - _P)
    gathered2 = _make_sc_gather(_V1, 2 * D, B)(t2, idx2)
    sel = (idx >= _V1).astype(jnp.int32).reshape(B, 1)
    return _mlp(gathered2, sel, W1, b1, W2, b2)


# speedup vs baseline: 2.3683x; 1.0070x over previous
import functools

import jax
import jax.numpy as jnp
from jax import lax
from jax.experimental import pallas as pl
from jax.experimental.pallas import tpu as pltpu
from jax.experimental.pallas import tpu_sc as plsc

_CH = 32
_CB = 8192
_V1 = 507904
_P = 499712


def _repack_body(xa_ref, xb_ref, o_ref):
    xa = xa_ref[...]
    xb = xb_ref[...]
    o_ref[...] = jnp.concatenate([xa.T, xb.T], axis=1)


def _repack(tableT):
    D, V = tableT.shape
    grid = _V1 // _CB
    return pl.pallas_call(
        _repack_body,
        grid=(grid,),
        in_specs=[
            pl.BlockSpec((D, _CB), lambda i: (0, i)),
            pl.BlockSpec((D, _CB), lambda i: (0, i + _P // _CB)),
        ],
        out_specs=pl.BlockSpec((_CB, 2 * D), lambda i: (i, 0)),
        out_shape=jax.ShapeDtypeStruct((_V1, 2 * D), jnp.float32),
    )(tableT, tableT)


def _make_sc_gather(V2, D2, B):
    info = plsc.get_sparse_core_info()
    NC, NS = info.num_cores, info.num_subcores
    NW = NC * NS
    assert B % (8 * NW) == 0 and D2 % info.num_lanes == 0
    b_per_w = B // NW
    mesh = plsc.VectorSubcoreMesh(core_axis_name="c", subcore_axis_name="s")

    @functools.partial(
        pl.kernel,
        mesh=mesh,
        out_type=jax.ShapeDtypeStruct((B, D2), jnp.float32),
        scratch_types=[
            pltpu.VMEM((b_per_w // 128, 128), jnp.int32),
            pltpu.VMEM((b_per_w, D2), jnp.float32),
            pltpu.SemaphoreType.DMA,
            pltpu.SemaphoreType.DMA,
        ],
    )
    def gather_k(table_hbm, idx_hbm, out_hbm, idx_v, rows_v, sem_idx, sem):
        wid = lax.axis_index("s") * NC + lax.axis_index("c")
        base = wid * b_per_w
        nj = b_per_w // 128
        for j in range(nj):
            pltpu.async_copy(
                idx_hbm.at[pl.ds(base + j * 128, 128)], idx_v.at[j], sem_idx
            )
        for j in range(nj):
            pltpu.make_async_copy(
                idx_hbm.at[pl.ds(base + j * 128, 128)], idx_v.at[j], sem_idx
            ).wait()
        for j in range(nj):
            pltpu.async_copy(
                table_hbm.at[idx_v.at[j]], rows_v.at[pl.ds(j * 128, 128)], sem
            )
        for j in range(nj):
            pltpu.make_async_copy(
                table_hbm.at[idx_v.at[j]], rows_v.at[pl.ds(j * 128, 128)], sem
            ).wait()
        pltpu.sync_copy(rows_v, out_hbm.at[pl.ds(base, b_per_w)])

    return gather_k


def _mlp_body(x2_ref, par_ref, w1_ref, b1_ref, w2_ref, b2_ref, o_ref):
    x2 = x2_ref[...]
    par = par_ref[...]
    D = w1_ref.shape[0]
    x = jnp.where(par == 0, x2[:, :D], x2[:, D:])
    h = jnp.dot(x, w1_ref[...], preferred_element_type=jnp.float32) + b1_ref[...]
    h = jnp.maximum(h, 0.0)
    y = jnp.dot(h, w2_ref[...], preferred_element_type=jnp.float32) + b2_ref[...]
    ss = jnp.sum(y * y, axis=-1, keepdims=True)
    o_ref[...] = y / jnp.maximum(jnp.sqrt(ss), 1e-12)


def _mlp(gathered2, parity, W1, b1, W2, b2, blk=2048):
    B, D2 = gathered2.shape
    D = D2 // 2
    H = W1.shape[1]
    O = W2.shape[1]
    return pl.pallas_call(
        _mlp_body,
        grid=(B // blk,),
        in_specs=[
            pl.BlockSpec((blk, D2), lambda i: (i, 0)),
            pl.BlockSpec((blk, 1), lambda i: (i, 0)),
            pl.BlockSpec((D, H), lambda i: (0, 0)),
            pl.BlockSpec((1, H), lambda i: (0, 0)),
            pl.BlockSpec((H, O), lambda i: (0, 0)),
            pl.BlockSpec((1, O), lambda i: (0, 0)),
        ],
        out_specs=pl.BlockSpec((blk, O), lambda i: (i, 0)),
        out_shape=jax.ShapeDtypeStruct((B, O), jnp.float32),
    )(gathered2, parity, W1, b1.reshape(1, H), W2, b2.reshape(1, O))


def kernel(indices, table, W1, b1, W2, b2):
    idx = indices.astype(jnp.int32)
    B = idx.shape[0]
    V, D = table.shape
    t2 = _repack(table.T)
    idx2 = jnp.where(idx < _V1, idx, idx - _P)
    gathered2 = _make_sc_gather(_V1, 2 * D, B)(t2, idx2)
    sel = (idx >= _V1).astype(jnp.int32).reshape(B, 1)
    return _mlp(gathered2, sel, W1, b1, W2, b2)
